# baseline (speedup 1.0000x reference)
import functools

import jax
import jax.numpy as jnp
import numpy as np
from jax import lax
from jax.experimental import pallas as pl
from jax.experimental.pallas import tpu as pltpu
from jax.experimental.pallas import tpu_sc as plsc

_K = 16
_EPS = 1e-5

_NC, _NS = 2, 16
_NW = _NC * _NS
_CHUNK = 128
_CPR = 4
_RPW = 10
_GD = 128
_RPR = _CPR * _CHUNK
_BPAD = _NW * _RPW * _RPR


def _sc_gather(table, idx_flat):
    mesh = plsc.VectorSubcoreMesh(core_axis_name="c", subcore_axis_name="s")

    @functools.partial(
        pl.kernel, mesh=mesh,
        out_type=jax.ShapeDtypeStruct((_BPAD, _GD), jnp.float32),
        scratch_types=[
            pltpu.VMEM((_RPR,), jnp.int32),
            pltpu.VMEM((_RPR, _GD), jnp.float32),
            pltpu.SemaphoreType.DMA,
        ],
    )
    def k(table_hbm, idx_hbm, out_hbm, idx_v, rows_v, sem):
        wid = lax.axis_index("s") * _NC + lax.axis_index("c")
        wbase = wid * (_RPW * _RPR)

        def round_body(r):
            base = wbase + r * _RPR
            pltpu.sync_copy(idx_hbm.at[pl.ds(base, _RPR)], idx_v)
            copies = []
            for c in range(_CPR):
                copies.append(pltpu.async_copy(
                    table_hbm.at[idx_v.at[pl.ds(c * _CHUNK, _CHUNK)]],
                    rows_v.at[pl.ds(c * _CHUNK, _CHUNK)], sem))
            for cp in copies:
                cp.wait()
            pltpu.sync_copy(rows_v, out_hbm.at[pl.ds(base, _RPR)])

        pl.loop(0, _RPW)(round_body)

    return k(table, idx_flat)


def _gather_rows(table_nc, idx_bnk):
    N, C = table_nc.shape
    table_p = jnp.pad(table_nc, ((0, 0), (0, _GD - C)))
    idx_flat = idx_bnk.reshape(-1)
    idx_flat = jnp.pad(idx_flat, (0, _BPAD - idx_flat.shape[0]))
    return _sc_gather(table_p, idx_flat)


_KNN_R = 256
_CAND = 2048
_ROWS_W = 320


def _dist_thr_kernel(xr_ref, xct_ref, d_ref, thr_ref):
    npad = xct_ref.shape[1]
    xr = xr_ref[:]
    xct = xct_ref[:]
    sqr = jnp.sum(xr * xr, axis=1, keepdims=True)
    sqc = jnp.sum(xct * xct, axis=0, keepdims=True)
    d = sqr + sqc - 2.0 * jnp.dot(xr, xct, preferred_element_type=jnp.float32)
    col = lax.broadcasted_iota(jnp.int32, d.shape, 1)
    d = jnp.where(col >= 10000, jnp.inf, d)
    d_ref[:] = d
    g = npad // 16
    thr = jnp.min(d[:, :g], axis=1)
    for c in range(1, 16):
        thr = jnp.maximum(thr, jnp.min(d[:, c * g:(c + 1) * g], axis=1))
    thr_ref[:] = thr


def _sc_knn_select(dist, thr):
    npad = dist.shape[0]
    nchunks = npad // 16
    mesh = plsc.VectorSubcoreMesh(core_axis_name="c", subcore_axis_name="s")

    @functools.partial(
        pl.kernel, mesh=mesh,
        out_type=jax.ShapeDtypeStruct((npad, _K), jnp.int32),
        scratch_types=[
            pltpu.VMEM((2, npad), jnp.float32),
            pltpu.VMEM((_ROWS_W + 16,), jnp.float32),
            pltpu.VMEM((_CAND + 16,), jnp.float32),
            pltpu.VMEM((_CAND + 16,), jnp.int32),
            pltpu.VMEM((_ROWS_W, _K), jnp.int32),
            pltpu.SemaphoreType.DMA,
            pltpu.SemaphoreType.DMA,
        ],
        compiler_params=pltpu.CompilerParams(needs_layout_passes=False),
    )
    def k(dist_hbm, thr_hbm, nn_hbm, rowbuf, thr_v, cv, ci, nn_v, sem_a, sem_b):
        wid = lax.axis_index("s") * _NC + lax.axis_index("c")
        rbase = wid * _ROWS_W
        pltpu.sync_copy(thr_hbm.at[pl.ds(rbase, _ROWS_W)],
                        thr_v.at[pl.ds(0, _ROWS_W)])
        pltpu.async_copy(dist_hbm.at[rbase], rowbuf.at[0], sem_a)
        pltpu.async_copy(dist_hbm.at[rbase + 1], rowbuf.at[1], sem_b)

        iota = lax.broadcasted_iota(jnp.int32, (16,), 0)
        zeros = jnp.zeros((16,), jnp.int32)
        inf = jnp.full((16,), jnp.inf, jnp.float32)

        def process_row(r, slot, sem):
            pltpu.make_async_copy(
                dist_hbm.at[rbase], rowbuf.at[slot], sem).wait()
            tv = jnp.broadcast_to(thr_v[pl.ds(r, 16)][0], (16,))

            def chunk(c, off):
                d = rowbuf[slot, pl.ds(c * 16, 16)]
                m = d <= tv
                mi = jnp.where(m, 1, 0).astype(jnp.int32)
                pos = jnp.minimum(plsc.cumsum(mi) + (off - 1), _CAND - 1)
                plsc.store_scatter(cv, [pos], d, mask=m)
                plsc.store_scatter(ci, [pos], iota + c * 16, mask=m)
                return pos[15] + 1

            ncand = lax.fori_loop(0, nchunks, chunk, jnp.int32(0))

            def merge_step(bv, bi, v, i):
                sv, si = plsc.sort_key_val(v, i)
                rv = lax.rev(sv, (0,))
                ri = lax.rev(si, (0,))
                take = bv <= rv
                lv = jnp.where(take, bv, rv)
                li = jnp.where(take, bi, ri)
                sv2, si2 = plsc.sort_key_val(lv, li)
                return (sv2, si2)

            def merge(c, carry):
                bv, bi = carry
                base = iota + c * 16
                v = cv[pl.ds(c * 16, 16)]
                i = ci[pl.ds(c * 16, 16)]
                v = jnp.where(base < ncand, v, jnp.inf)
                return merge_step(bv, bi, v, i)

            def fb_merge(c, carry):
                bv, bi = carry
                d = rowbuf[slot, pl.ds(c * 16, 16)]
                return merge_step(bv, bi, d, iota + c * 16)

            bv, bi = lax.cond(
                ncand >= _CAND,
                lambda: lax.fori_loop(0, nchunks, fb_merge, (inf, zeros)),
                lambda: lax.fori_loop(
                    0, (ncand + 15) // 16, merge, (inf, zeros)))
            plsc.store_scatter(nn_v, [jnp.full((16,), r, jnp.int32), iota], bi)

            @pl.when(r + 2 < _ROWS_W)
            def _():
                pltpu.async_copy(
                    dist_hbm.at[rbase + r + 2], rowbuf.at[slot], sem)

        def body(j):
            process_row(2 * j, 0, sem_a)
            process_row(2 * j + 1, 1, sem_b)

        pl.loop(0, _ROWS_W // 2)(body)
        pltpu.sync_copy(nn_v, nn_hbm.at[pl.ds(rbase, _ROWS_W)])

    return k(dist, thr)


def _dense_knn(x, k):
    N = x.shape[2]
    npad = ((N + _KNN_R - 1) // _KNN_R) * _KNN_R
    xt = jnp.transpose(x[0, :, :, 0], (1, 0))
    xtp = jnp.pad(xt, ((0, npad - N), (0, 5)))
    dist, thr = pl.pallas_call(
        _dist_thr_kernel,
        grid=(npad // _KNN_R,),
        in_specs=[
            pl.BlockSpec((_KNN_R, 8), lambda i: (i, 0)),
            pl.BlockSpec((8, npad), lambda i: (0, 0)),
        ],
        out_specs=[
            pl.BlockSpec((_KNN_R, npad), lambda i: (i, 0)),
            pl.BlockSpec((_KNN_R,), lambda i: (i,)),
        ],
        out_shape=[
            jax.ShapeDtypeStruct((npad, npad), jnp.float32),
            jax.ShapeDtypeStruct((npad,), jnp.float32),
        ],
    )(xtp, xtp.T)
    out = _sc_knn_select(dist, thr)
    return out[:N][None]


_EBLK = 2048
_NE = 10000 * _K
_NPADN = 10240


def _stats_rows(y, mask):
    ym = jnp.where(mask, y, 0.0)
    s0 = jnp.sum(ym, axis=0, keepdims=True)
    s1 = jnp.sum(ym * ym, axis=0, keepdims=True)
    z = jnp.zeros((6, y.shape[1]), jnp.float32)
    return jnp.concatenate([s0, s1, z], axis=0)


def _edge_mask(i, n):
    eid = i * _EBLK + lax.broadcasted_iota(jnp.int32, (_EBLK, 1), 0)
    return eid < n


def _edge1_kernel(xr_ref, gj_ref, wt_ref, b_ref, y_ref, s_ref):
    i = pl.program_id(0)
    xr = xr_ref[:]
    gj = gj_ref[:]
    eij = xr - gj
    e0 = jnp.concatenate([eij, xr, gj], axis=1)
    y = jnp.dot(e0, wt_ref[:],
                preferred_element_type=jnp.float32) + b_ref[:][None]
    y_ref[:] = y

    @pl.when(i == 0)
    def _():
        s_ref[:] = jnp.zeros_like(s_ref)
    s_ref[:] += _stats_rows(y, _edge_mask(i, _NE))


def _bn_mm_stats_kernel(y_ref, sc_ref, sh_ref, wt_ref, b_ref, o_ref, s_ref):
    i = pl.program_id(0)
    x = jnp.maximum(y_ref[:] * sc_ref[:][None] + sh_ref[:][None], 0.0)
    o = jnp.dot(x, wt_ref[:],
                preferred_element_type=jnp.float32) + b_ref[:][None]
    o_ref[:] = o

    @pl.when(i == 0)
    def _():
        s_ref[:] = jnp.zeros_like(s_ref)
    s_ref[:] += _stats_rows(o, _edge_mask(i, _NE))


def _edge4_kernel(y_ref, hr_ref, hj_ref, sc_ref, sh_ref, wt_ref, b_ref,
                  o_ref, s_ref):
    i = pl.program_id(0)
    e1 = jnp.maximum(y_ref[:] * sc_ref[:][None] + sh_ref[:][None], 0.0)
    e = jnp.concatenate([e1, hr_ref[:], hj_ref[:]], axis=1)
    o = jnp.dot(e, wt_ref[:],
                preferred_element_type=jnp.float32) + b_ref[:][None]
    o_ref[:] = o

    @pl.when(i == 0)
    def _():
        s_ref[:] = jnp.zeros_like(s_ref)
    s_ref[:] += _stats_rows(o, _edge_mask(i, _NE))


def _ksum_kernel(y_ref, sc_ref, sh_ref, m_ref):
    e = jnp.maximum(y_ref[:] * sc_ref[:][None] + sh_ref[:][None], 0.0)
    c = e.shape[1]
    m_ref[:] = jnp.sum(e.reshape(_EBLK // _K, _K, c), axis=1)


def _bn_finalize(s, cnt, gamma, beta):
    mean = s[0] / cnt
    var = jnp.maximum(s[1] / cnt - mean * mean, 0.0)
    scale = gamma * jax.lax.rsqrt(var + _EPS)
    shift = beta - mean * scale
    return scale, shift


def _node_mask_bn_relu(y, g, b, n_valid):
    row = lax.broadcasted_iota(jnp.int32, (y.shape[0], 1), 0)
    m = row < n_valid
    ym = jnp.where(m, y, 0.0)
    mean = jnp.sum(ym, axis=0, keepdims=True) / n_valid
    var = jnp.sum(jnp.where(m, (y - mean) ** 2, 0.0), axis=0,
                  keepdims=True) / n_valid
    yn = (y - mean) * jax.lax.rsqrt(var + _EPS) * g[:][None] + b[:][None]
    return jnp.maximum(yn, 0.0)


def _node_kernel(xa_ref, xb_ref, w1_ref, b1_ref, g1_ref, bt1_ref,
                 w2_ref, b2_ref, g2_ref, bt2_ref, h_ref):
    x = jnp.concatenate([xa_ref[:], xb_ref[:]], axis=1)
    y = jnp.dot(x, w1_ref[:],
                preferred_element_type=jnp.float32) + b1_ref[:][None]
    y = _node_mask_bn_relu(y, g1_ref, bt1_ref, 10000)
    y = jnp.dot(y, w2_ref[:],
                preferred_element_type=jnp.float32) + b2_ref[:][None]
    h_ref[:] = _node_mask_bn_relu(y, g2_ref, bt2_ref, 10000)


def _pred_kernel(h1_ref, h2_ref, fw_ref, fb_ref, fg_ref, fbt_ref,
                 w1_ref, b1_ref, g1_ref, bt1_ref,
                 w2_ref, b2_ref, g2_ref, bt2_ref,
                 w3_ref, b3_ref, out_ref):
    feats = jnp.concatenate([h1_ref[:], h2_ref[:]], axis=1)
    fus = jnp.dot(feats, fw_ref[:],
                  preferred_element_type=jnp.float32) + fb_ref[:][None]
    fus = _node_mask_bn_relu(fus, fg_ref, fbt_ref, 10000)
    row = lax.broadcasted_iota(jnp.int32, (feats.shape[0], 1), 0)
    fmax = jnp.max(jnp.where(row < 10000, fus, -jnp.inf), axis=0,
                   keepdims=True)
    x = jnp.concatenate(
        [jnp.broadcast_to(fmax, (feats.shape[0], fmax.shape[1])), feats],
        axis=1)
    x = jnp.dot(x, w1_ref[:],
                preferred_element_type=jnp.float32) + b1_ref[:][None]
    x = _node_mask_bn_relu(x, g1_ref, bt1_ref, 10000)
    x = jnp.dot(x, w2_ref[:],
                preferred_element_type=jnp.float32) + b2_ref[:][None]
    x = _node_mask_bn_relu(x, g2_ref, bt2_ref, 10000)
    out_ref[:] = jnp.dot(x, w3_ref[:],
                         preferred_element_type=jnp.float32) + b3_ref[:][None]


def _edge_call(body, n_in_blocked, consts, cout, extra_out=None):
    grid = _BPAD // _EBLK
    in_specs = []
    args = []
    for a in n_in_blocked:
        in_specs.append(pl.BlockSpec((_EBLK, a.shape[1]), lambda i: (i, 0)))
        args.append(a)
    for cst in consts:
        zmap = (lambda r: lambda i: (0,) * r)(len(cst.shape))
        in_specs.append(pl.BlockSpec(cst.shape, zmap))
        args.append(cst)
    out_specs = [pl.BlockSpec((_EBLK, cout), lambda i: (i, 0)),
                 pl.BlockSpec((8, cout), lambda i: (0, 0))]
    out_shape = [jax.ShapeDtypeStruct((_BPAD, cout), jnp.float32),
                 jax.ShapeDtypeStruct((8, cout), jnp.float32)]
    return pl.pallas_call(
        body, grid=(grid,), in_specs=in_specs,
        out_specs=out_specs, out_shape=out_shape)(*args)


def _ksum_call(y, scale, shift):
    grid = _BPAD // _EBLK
    c = y.shape[1]
    return pl.pallas_call(
        _ksum_kernel, grid=(grid,),
        in_specs=[
            pl.BlockSpec((_EBLK, c), lambda i: (i, 0)),
            pl.BlockSpec((c,), lambda i: (0,)),
            pl.BlockSpec((c,), lambda i: (0,)),
        ],
        out_specs=pl.BlockSpec((_EBLK // _K, c), lambda i: (i, 0)),
        out_shape=jax.ShapeDtypeStruct((_BPAD // _K, c), jnp.float32),
    )(y, scale, shift)


def _node_call(xa, xb, p1, p2, cout, c1):
    (w1, b1, g1, bt1), (w2, b2, g2, bt2) = p1, p2
    cin = xa.shape[1] + xb.shape[1]
    w1p = jnp.zeros((cin, w1.shape[0]), jnp.float32)
    w1p = w1p.at[:c1].set(w1[:, :c1].T)
    w1p = w1p.at[xa.shape[1]:].set(w1[:, c1:].T)
    return pl.pallas_call(
        _node_kernel,
        out_shape=jax.ShapeDtypeStruct((_NPADN, cout), jnp.float32),
    )(xa, xb, w1p, b1, g1, bt1, w2.T, b2, g2, bt2)


def kernel(inputs, params):
    inputs = inputs[:, :6]
    nn_idx = _dense_knn(inputs[:, 0:3], _K)

    x6 = inputs[0, :, :, 0].T
    x6p = jnp.pad(x6, ((0, _NPADN - x6.shape[0]), (0, 10)))
    g6 = _gather_rows(x6, nn_idx)[:, :16]
    x6r = jnp.pad(jnp.repeat(x6, _K, axis=0),
                  ((0, _BPAD - _NE), (0, 10)))

    (w1, b1, g1, bt1), (w2, b2, g2, bt2) = params['head_edge']
    w1p = jnp.zeros((48, 16), jnp.float32)
    w1p = w1p.at[0:3].set(w1[:, 0:3].T)
    w1p = w1p.at[16:22].set(w1[:, 3:9].T)
    w1p = w1p.at[32:38].set(w1[:, 9:15].T)
    y1, s1 = _edge_call(_edge1_kernel, [x6r, g6], [w1p, b1], 16)
    sc1, sh1 = _bn_finalize(s1, _NE, g1, bt1)
    y2, s2 = _edge_call(_bn_mm_stats_kernel, [y1], [sc1, sh1, w2.T, b2], 32)
    sc2, sh2 = _bn_finalize(s2, _NE, g2, bt2)
    m1 = _ksum_call(y2, sc2, sh2)

    h1 = _node_call(x6p, m1, *params['head_node'], cout=32, c1=6)

    g32 = _gather_rows(h1[:10000], nn_idx)[:, :32]
    h1r = jnp.pad(jnp.repeat(h1[:10000], _K, axis=0),
                  ((0, _BPAD - _NE), (0, 0)))
    (w3, b3, g3, bt3), (w4, b4, g4, bt4) = params['b1_edge']
    y3, s3 = _edge_call(_edge4_kernel, [y2, h1r, g32],
                        [sc2, sh2, w3.T, b3], 24)
    sc3, sh3 = _bn_finalize(s3, _NE, g3, bt3)
    y4, s4 = _edge_call(_bn_mm_stats_kernel, [y3], [sc3, sh3, w4.T, b4], 64)
    sc4, sh4 = _bn_finalize(s4, _NE, g4, bt4)
    m2 = _ksum_call(y4, sc4, sh4)

    h2 = _node_call(h1, m2, *params['b1_node'], cout=64, c1=32)

    fw, fb, fg, fbt = params['fusion']
    pw1, pb1, pg1, pbt1 = params['pred1']
    pw2, pb2, pg2, pbt2 = params['pred2']
    pw3, pb3, _, _ = params['pred3']
    w3p = jnp.zeros((128, 16), jnp.float32).at[:, :13].set(pw3.T)
    b3p = jnp.zeros((16,), jnp.float32).at[:13].set(pb3)
    out = pl.pallas_call(
        _pred_kernel,
        out_shape=jax.ShapeDtypeStruct((_NPADN, 16), jnp.float32),
    )(h1, h2, fw.T, fb, fg, fbt,
      pw1.T, pb1, pg1, pbt1, pw2.T, pb2, pg2, pbt2, w3p, b3p)
    return out[:10000, :13].T[None]


# speedup vs baseline: 1.2886x; 1.2886x over previous
import functools

import jax
import jax.numpy as jnp
import numpy as np
from jax import lax
from jax.experimental import pallas as pl
from jax.experimental.pallas import tpu as pltpu
from jax.experimental.pallas import tpu_sc as plsc

_K = 16
_EPS = 1e-5

_NC, _NS = 2, 16
_NW = _NC * _NS
_CHUNK = 128
_CPR = 4
_RPW = 10
_GD = 128
_RPR = _CPR * _CHUNK
_BPAD = _NW * _RPW * _RPR


def _sc_gather(table, idx_flat):
    mesh = plsc.VectorSubcoreMesh(core_axis_name="c", subcore_axis_name="s")

    @functools.partial(
        pl.kernel, mesh=mesh,
        out_type=jax.ShapeDtypeStruct((_BPAD, _GD), jnp.float32),
        scratch_types=[
            pltpu.VMEM((_RPR,), jnp.int32),
            pltpu.VMEM((_RPR, _GD), jnp.float32),
            pltpu.SemaphoreType.DMA,
        ],
    )
    def k(table_hbm, idx_hbm, out_hbm, idx_v, rows_v, sem):
        wid = lax.axis_index("s") * _NC + lax.axis_index("c")
        wbase = wid * (_RPW * _RPR)

        def round_body(r):
            base = wbase + r * _RPR
            pltpu.sync_copy(idx_hbm.at[pl.ds(base, _RPR)], idx_v)
            copies = []
            for c in range(_CPR):
                copies.append(pltpu.async_copy(
                    table_hbm.at[idx_v.at[pl.ds(c * _CHUNK, _CHUNK)]],
                    rows_v.at[pl.ds(c * _CHUNK, _CHUNK)], sem))
            for cp in copies:
                cp.wait()
            pltpu.sync_copy(rows_v, out_hbm.at[pl.ds(base, _RPR)])

        pl.loop(0, _RPW)(round_body)

    return k(table, idx_flat)


def _gather_rows(table_nc, idx_bnk):
    N, C = table_nc.shape
    table_p = jnp.pad(table_nc, ((0, 0), (0, _GD - C)))
    idx_flat = idx_bnk.reshape(-1)
    idx_flat = jnp.pad(idx_flat, (0, _BPAD - idx_flat.shape[0]))
    return _sc_gather(table_p, idx_flat)


_KNN_R = 256
_CAND = 2048
_ROWS_W = 320


def _dist_thr_kernel(xr_ref, xct_ref, d_ref, thr_ref):
    npad = xct_ref.shape[1]
    xr = xr_ref[:]
    xct = xct_ref[:]
    sqr = jnp.sum(xr * xr, axis=1, keepdims=True)
    sqc = jnp.sum(xct * xct, axis=0, keepdims=True)
    d = sqr + sqc - 2.0 * jnp.dot(xr, xct, preferred_element_type=jnp.float32)
    col = lax.broadcasted_iota(jnp.int32, d.shape, 1)
    d = jnp.where(col >= 10000, jnp.inf, d)
    d_ref[:] = d
    g = npad // 16
    thr = jnp.min(d[:, :g], axis=1)
    for c in range(1, 16):
        thr = jnp.maximum(thr, jnp.min(d[:, c * g:(c + 1) * g], axis=1))
    thr_ref[:] = thr


def _sc_knn_select(dist, thr):
    npad = dist.shape[0]
    nchunks = npad // 16
    mesh = plsc.VectorSubcoreMesh(core_axis_name="c", subcore_axis_name="s")

    @functools.partial(
        pl.kernel, mesh=mesh,
        out_type=jax.ShapeDtypeStruct((npad, _K), jnp.int32),
        scratch_types=[
            pltpu.VMEM((2, npad), jnp.float32),
            pltpu.VMEM((_ROWS_W + 16,), jnp.float32),
            pltpu.VMEM((_CAND + 16,), jnp.float32),
            pltpu.VMEM((_CAND + 16,), jnp.int32),
            pltpu.VMEM((_ROWS_W, _K), jnp.int32),
            pltpu.SemaphoreType.DMA,
            pltpu.SemaphoreType.DMA,
        ],
        compiler_params=pltpu.CompilerParams(needs_layout_passes=False),
    )
    def k(dist_hbm, thr_hbm, nn_hbm, rowbuf, thr_v, cv, ci, nn_v, sem_a, sem_b):
        wid = lax.axis_index("s") * _NC + lax.axis_index("c")
        rbase = wid * _ROWS_W
        pltpu.sync_copy(thr_hbm.at[pl.ds(rbase, _ROWS_W)],
                        thr_v.at[pl.ds(0, _ROWS_W)])
        pltpu.async_copy(dist_hbm.at[rbase], rowbuf.at[0], sem_a)
        pltpu.async_copy(dist_hbm.at[rbase + 1], rowbuf.at[1], sem_b)

        iota = lax.broadcasted_iota(jnp.int32, (16,), 0)
        zeros = jnp.zeros((16,), jnp.int32)
        inf = jnp.full((16,), jnp.inf, jnp.float32)

        def process_row(r, slot, sem):
            pltpu.make_async_copy(
                dist_hbm.at[rbase], rowbuf.at[slot], sem).wait()
            tv = jnp.broadcast_to(thr_v[pl.ds(r, 16)][0], (16,))

            def chunk(c, off):
                d = rowbuf[slot, pl.ds(c * 16, 16)]
                m = d <= tv
                mi = jnp.where(m, 1, 0).astype(jnp.int32)
                pos = jnp.minimum(plsc.cumsum(mi) + (off - 1), _CAND - 1)
                plsc.store_scatter(cv, [pos], d, mask=m)
                plsc.store_scatter(ci, [pos], iota + c * 16, mask=m)
                return pos[15] + 1

            ncand = lax.fori_loop(0, nchunks, chunk, jnp.int32(0))

            def merge_step(bv, bi, v, i):
                sv, si = plsc.sort_key_val(v, i)
                rv = lax.rev(sv, (0,))
                ri = lax.rev(si, (0,))
                take = bv <= rv
                lv = jnp.where(take, bv, rv)
                li = jnp.where(take, bi, ri)
                sv2, si2 = plsc.sort_key_val(lv, li)
                return (sv2, si2)

            def merge(c, carry):
                bv, bi = carry
                base = iota + c * 16
                v = cv[pl.ds(c * 16, 16)]
                i = ci[pl.ds(c * 16, 16)]
                v = jnp.where(base < ncand, v, jnp.inf)
                return merge_step(bv, bi, v, i)

            def fb_merge(c, carry):
                bv, bi = carry
                d = rowbuf[slot, pl.ds(c * 16, 16)]
                return merge_step(bv, bi, d, iota + c * 16)

            bv, bi = lax.cond(
                ncand >= _CAND,
                lambda: lax.fori_loop(0, nchunks, fb_merge, (inf, zeros)),
                lambda: lax.fori_loop(
                    0, (ncand + 15) // 16, merge, (inf, zeros)))
            plsc.store_scatter(nn_v, [jnp.full((16,), r, jnp.int32), iota], bi)

            @pl.when(r + 2 < _ROWS_W)
            def _():
                pltpu.async_copy(
                    dist_hbm.at[rbase + r + 2], rowbuf.at[slot], sem)

        def body(j):
            process_row(2 * j, 0, sem_a)
            process_row(2 * j + 1, 1, sem_b)

        pl.loop(0, _ROWS_W // 2)(body)
        pltpu.sync_copy(nn_v, nn_hbm.at[pl.ds(rbase, _ROWS_W)])

    return k(dist, thr)


def _knn_argmin_kernel(xr_ref, xct_ref, out_ref):
    npad = xct_ref.shape[1]
    xr = xr_ref[:]
    xct = xct_ref[:]
    sqr = jnp.sum(xr * xr, axis=1, keepdims=True)
    sqc = jnp.sum(xct * xct, axis=0, keepdims=True)
    d = sqr + sqc - 2.0 * jnp.dot(xr, xct, preferred_element_type=jnp.float32)
    col = lax.broadcasted_iota(jnp.int32, d.shape, 1)
    d = jnp.where(col >= 10000, jnp.inf, d)
    cols = []
    for _ in range(_K):
        idx = jnp.argmin(d, axis=1).astype(jnp.int32)
        cols.append(idx)
        d = jnp.where(col == idx[:, None], jnp.inf, d)
    out_ref[:] = jnp.stack(cols, axis=1)


def _dense_knn_argmin(x, k):
    N = x.shape[2]
    npad = ((N + _KNN_R - 1) // _KNN_R) * _KNN_R
    xt = jnp.transpose(x[0, :, :, 0], (1, 0))
    xtp = jnp.pad(xt, ((0, npad - N), (0, 5)))
    out = pl.pallas_call(
        _knn_argmin_kernel,
        grid=(npad // _KNN_R,),
        in_specs=[
            pl.BlockSpec((_KNN_R, 8), lambda i: (i, 0)),
            pl.BlockSpec((8, npad), lambda i: (0, 0)),
        ],
        out_specs=pl.BlockSpec((_KNN_R, _K), lambda i: (i, 0)),
        out_shape=jax.ShapeDtypeStruct((npad, _K), jnp.int32),
    )(xtp, xtp.T)
    return out[:N][None]


def _dense_knn(x, k):
    N = x.shape[2]
    npad = ((N + _KNN_R - 1) // _KNN_R) * _KNN_R
    xt = jnp.transpose(x[0, :, :, 0], (1, 0))
    xtp = jnp.pad(xt, ((0, npad - N), (0, 5)))
    dist, thr = pl.pallas_call(
        _dist_thr_kernel,
        grid=(npad // _KNN_R,),
        in_specs=[
            pl.BlockSpec((_KNN_R, 8), lambda i: (i, 0)),
            pl.BlockSpec((8, npad), lambda i: (0, 0)),
        ],
        out_specs=[
            pl.BlockSpec((_KNN_R, npad), lambda i: (i, 0)),
            pl.BlockSpec((_KNN_R,), lambda i: (i,)),
        ],
        out_shape=[
            jax.ShapeDtypeStruct((npad, npad), jnp.float32),
            jax.ShapeDtypeStruct((npad,), jnp.float32),
        ],
    )(xtp, xtp.T)
    out = _sc_knn_select(dist, thr)
    return out[:N][None]


_EBLK = 2048
_NE = 10000 * _K
_NPADN = 10240


def _stats_rows(y, mask):
    ym = jnp.where(mask, y, 0.0)
    s0 = jnp.sum(ym, axis=0, keepdims=True)
    s1 = jnp.sum(ym * ym, axis=0, keepdims=True)
    z = jnp.zeros((6, y.shape[1]), jnp.float32)
    return jnp.concatenate([s0, s1, z], axis=0)


def _edge_mask(i, n):
    eid = i * _EBLK + lax.broadcasted_iota(jnp.int32, (_EBLK, 1), 0)
    return eid < n


def _edge1_kernel(xr_ref, gj_ref, wt_ref, b_ref, y_ref, s_ref):
    i = pl.program_id(0)
    xr = xr_ref[:]
    gj = gj_ref[:]
    eij = xr - gj
    e0 = jnp.concatenate([eij, xr, gj], axis=1)
    y = jnp.dot(e0, wt_ref[:],
                preferred_element_type=jnp.float32) + b_ref[:][None]
    y_ref[:] = y

    @pl.when(i == 0)
    def _():
        s_ref[:] = jnp.zeros_like(s_ref)
    s_ref[:] += _stats_rows(y, _edge_mask(i, _NE))


def _bn_mm_stats_kernel(y_ref, sc_ref, sh_ref, wt_ref, b_ref, o_ref, s_ref):
    i = pl.program_id(0)
    x = jnp.maximum(y_ref[:] * sc_ref[:][None] + sh_ref[:][None], 0.0)
    o = jnp.dot(x, wt_ref[:],
                preferred_element_type=jnp.float32) + b_ref[:][None]
    o_ref[:] = o

    @pl.when(i == 0)
    def _():
        s_ref[:] = jnp.zeros_like(s_ref)
    s_ref[:] += _stats_rows(o, _edge_mask(i, _NE))


def _edge4_kernel(y_ref, hr_ref, hj_ref, sc_ref, sh_ref, wt_ref, b_ref,
                  o_ref, s_ref):
    i = pl.program_id(0)
    e1 = jnp.maximum(y_ref[:] * sc_ref[:][None] + sh_ref[:][None], 0.0)
    e = jnp.concatenate([e1, hr_ref[:], hj_ref[:]], axis=1)
    o = jnp.dot(e, wt_ref[:],
                preferred_element_type=jnp.float32) + b_ref[:][None]
    o_ref[:] = o

    @pl.when(i == 0)
    def _():
        s_ref[:] = jnp.zeros_like(s_ref)
    s_ref[:] += _stats_rows(o, _edge_mask(i, _NE))


def _ksum_kernel(y_ref, sc_ref, sh_ref, m_ref):
    e = jnp.maximum(y_ref[:] * sc_ref[:][None] + sh_ref[:][None], 0.0)
    c = e.shape[1]
    m_ref[:] = jnp.sum(e.reshape(_EBLK // _K, _K, c), axis=1)


def _bn_finalize(s, cnt, gamma, beta):
    mean = s[0] / cnt
    var = jnp.maximum(s[1] / cnt - mean * mean, 0.0)
    scale = gamma * jax.lax.rsqrt(var + _EPS)
    shift = beta - mean * scale
    return scale, shift


def _node_mask_bn_relu(y, g, b, n_valid):
    row = lax.broadcasted_iota(jnp.int32, (y.shape[0], 1), 0)
    m = row < n_valid
    ym = jnp.where(m, y, 0.0)
    mean = jnp.sum(ym, axis=0, keepdims=True) / n_valid
    var = jnp.sum(jnp.where(m, (y - mean) ** 2, 0.0), axis=0,
                  keepdims=True) / n_valid
    yn = (y - mean) * jax.lax.rsqrt(var + _EPS) * g[:][None] + b[:][None]
    return jnp.maximum(yn, 0.0)


def _node_kernel(xa_ref, xb_ref, w1_ref, b1_ref, g1_ref, bt1_ref,
                 w2_ref, b2_ref, g2_ref, bt2_ref, h_ref):
    x = jnp.concatenate([xa_ref[:], xb_ref[:]], axis=1)
    y = jnp.dot(x, w1_ref[:],
                preferred_element_type=jnp.float32) + b1_ref[:][None]
    y = _node_mask_bn_relu(y, g1_ref, bt1_ref, 10000)
    y = jnp.dot(y, w2_ref[:],
                preferred_element_type=jnp.float32) + b2_ref[:][None]
    h_ref[:] = _node_mask_bn_relu(y, g2_ref, bt2_ref, 10000)


def _pred_kernel(h1_ref, h2_ref, fw_ref, fb_ref, fg_ref, fbt_ref,
                 w1_ref, b1_ref, g1_ref, bt1_ref,
                 w2_ref, b2_ref, g2_ref, bt2_ref,
                 w3_ref, b3_ref, out_ref):
    feats = jnp.concatenate([h1_ref[:], h2_ref[:]], axis=1)
    fus = jnp.dot(feats, fw_ref[:],
                  preferred_element_type=jnp.float32) + fb_ref[:][None]
    fus = _node_mask_bn_relu(fus, fg_ref, fbt_ref, 10000)
    row = lax.broadcasted_iota(jnp.int32, (feats.shape[0], 1), 0)
    fmax = jnp.max(jnp.where(row < 10000, fus, -jnp.inf), axis=0,
                   keepdims=True)
    x = jnp.concatenate(
        [jnp.broadcast_to(fmax, (feats.shape[0], fmax.shape[1])), feats],
        axis=1)
    x = jnp.dot(x, w1_ref[:],
                preferred_element_type=jnp.float32) + b1_ref[:][None]
    x = _node_mask_bn_relu(x, g1_ref, bt1_ref, 10000)
    x = jnp.dot(x, w2_ref[:],
                preferred_element_type=jnp.float32) + b2_ref[:][None]
    x = _node_mask_bn_relu(x, g2_ref, bt2_ref, 10000)
    out_ref[:] = jnp.dot(x, w3_ref[:],
                         preferred_element_type=jnp.float32) + b3_ref[:][None]


def _edge_call(body, n_in_blocked, consts, cout, extra_out=None):
    grid = _BPAD // _EBLK
    in_specs = []
    args = []
    for a in n_in_blocked:
        in_specs.append(pl.BlockSpec((_EBLK, a.shape[1]), lambda i: (i, 0)))
        args.append(a)
    for cst in consts:
        zmap = (lambda r: lambda i: (0,) * r)(len(cst.shape))
        in_specs.append(pl.BlockSpec(cst.shape, zmap))
        args.append(cst)
    out_specs = [pl.BlockSpec((_EBLK, cout), lambda i: (i, 0)),
                 pl.BlockSpec((8, cout), lambda i: (0, 0))]
    out_shape = [jax.ShapeDtypeStruct((_BPAD, cout), jnp.float32),
                 jax.ShapeDtypeStruct((8, cout), jnp.float32)]
    return pl.pallas_call(
        body, grid=(grid,), in_specs=in_specs,
        out_specs=out_specs, out_shape=out_shape)(*args)


def _ksum_call(y, scale, shift):
    grid = _BPAD // _EBLK
    c = y.shape[1]
    return pl.pallas_call(
        _ksum_kernel, grid=(grid,),
        in_specs=[
            pl.BlockSpec((_EBLK, c), lambda i: (i, 0)),
            pl.BlockSpec((c,), lambda i: (0,)),
            pl.BlockSpec((c,), lambda i: (0,)),
        ],
        out_specs=pl.BlockSpec((_EBLK // _K, c), lambda i: (i, 0)),
        out_shape=jax.ShapeDtypeStruct((_BPAD // _K, c), jnp.float32),
    )(y, scale, shift)


def _node_call(xa, xb, p1, p2, cout, c1):
    (w1, b1, g1, bt1), (w2, b2, g2, bt2) = p1, p2
    cin = xa.shape[1] + xb.shape[1]
    w1p = jnp.zeros((cin, w1.shape[0]), jnp.float32)
    w1p = w1p.at[:c1].set(w1[:, :c1].T)
    w1p = w1p.at[xa.shape[1]:].set(w1[:, c1:].T)
    return pl.pallas_call(
        _node_kernel,
        out_shape=jax.ShapeDtypeStruct((_NPADN, cout), jnp.float32),
    )(xa, xb, w1p, b1, g1, bt1, w2.T, b2, g2, bt2)


def kernel(inputs, params):
    inputs = inputs[:, :6]
    nn_idx = _dense_knn_argmin(inputs[:, 0:3], _K)

    x6 = inputs[0, :, :, 0].T
    x6p = jnp.pad(x6, ((0, _NPADN - x6.shape[0]), (0, 10)))
    g6 = _gather_rows(x6, nn_idx)[:, :16]
    x6r = jnp.pad(jnp.repeat(x6, _K, axis=0),
                  ((0, _BPAD - _NE), (0, 10)))

    (w1, b1, g1, bt1), (w2, b2, g2, bt2) = params['head_edge']
    w1p = jnp.zeros((48, 16), jnp.float32)
    w1p = w1p.at[0:3].set(w1[:, 0:3].T)
    w1p = w1p.at[16:22].set(w1[:, 3:9].T)
    w1p = w1p.at[32:38].set(w1[:, 9:15].T)
    y1, s1 = _edge_call(_edge1_kernel, [x6r, g6], [w1p, b1], 16)
    sc1, sh1 = _bn_finalize(s1, _NE, g1, bt1)
    y2, s2 = _edge_call(_bn_mm_stats_kernel, [y1], [sc1, sh1, w2.T, b2], 32)
    sc2, sh2 = _bn_finalize(s2, _NE, g2, bt2)
    m1 = _ksum_call(y2, sc2, sh2)

    h1 = _node_call(x6p, m1, *params['head_node'], cout=32, c1=6)

    g32 = _gather_rows(h1[:10000], nn_idx)[:, :32]
    h1r = jnp.pad(jnp.repeat(h1[:10000], _K, axis=0),
                  ((0, _BPAD - _NE), (0, 0)))
    (w3, b3, g3, bt3), (w4, b4, g4, bt4) = params['b1_edge']
    y3, s3 = _edge_call(_edge4_kernel, [y2, h1r, g32],
                        [sc2, sh2, w3.T, b3], 24)
    sc3, sh3 = _bn_finalize(s3, _NE, g3, bt3)
    y4, s4 = _edge_call(_bn_mm_stats_kernel, [y3], [sc3, sh3, w4.T, b4], 64)
    sc4, sh4 = _bn_finalize(s4, _NE, g4, bt4)
    m2 = _ksum_call(y4, sc4, sh4)

    h2 = _node_call(h1, m2, *params['b1_node'], cout=64, c1=32)

    fw, fb, fg, fbt = params['fusion']
    pw1, pb1, pg1, pbt1 = params['pred1']
    pw2, pb2, pg2, pbt2 = params['pred2']
    pw3, pb3, _, _ = params['pred3']
    w3p = jnp.zeros((128, 16), jnp.float32).at[:, :13].set(pw3.T)
    b3p = jnp.zeros((16,), jnp.float32).at[:13].set(pb3)
    out = pl.pallas_call(
        _pred_kernel,
        out_shape=jax.ShapeDtypeStruct((_NPADN, 16), jnp.float32),
    )(h1, h2, fw.T, fb, fg, fbt,
      pw1.T, pb1, pg1, pbt1, pw2.T, pb2, pg2, pbt2, w3p, b3p)
    return out[:10000, :13].T[None]
